# Initial kernel scaffold; baseline (speedup 1.0000x reference)
import jax
import jax.numpy as jnp
from jax.experimental import pallas as pl


def kernel(cat_ids, table):
    raise NotImplementedError("write your pallas kernel here")


# speedup vs baseline: 2.7535x; 2.7535x over previous
import functools

import jax
import jax.numpy as jnp
from jax import lax
from jax.experimental import pallas as pl
from jax.experimental.pallas import tpu as pltpu
from jax.experimental.pallas import tpu_sc as plsc

EMB_DIM = 64
IDX_W = 128
GATHERS = 4
CHUNK = IDX_W * GATHERS


@functools.cache
def _gather_call(B2: int):
    info = plsc.get_sparse_core_info()
    NC, NS = info.num_cores, info.num_subcores
    NW = NC * NS
    b_per_w = B2 // NW
    n_iter = b_per_w // CHUNK
    assert b_per_w % CHUNK == 0
    mesh = plsc.VectorSubcoreMesh(core_axis_name="c", subcore_axis_name="s")

    @functools.partial(
        pl.kernel,
        mesh=mesh,
        out_type=jax.ShapeDtypeStruct((B2, 2 * EMB_DIM), jnp.float32),
        scratch_types=[
            pltpu.VMEM((CHUNK,), jnp.int32),
            pltpu.VMEM((CHUNK, 2 * EMB_DIM), jnp.float32),
            pltpu.SemaphoreType.DMA,
        ],
    )
    def k(table2_hbm, idx_hbm, out_hbm, idx_v, rows_v, sem):
        wid = lax.axis_index("s") * NC + lax.axis_index("c")
        base = wid * b_per_w

        def body(i, carry):
            row0 = base + i * CHUNK
            pltpu.sync_copy(idx_hbm.at[pl.ds(row0, CHUNK)], idx_v)
            copies = [
                pltpu.async_copy(
                    table2_hbm.at[idx_v.at[pl.ds(j * IDX_W, IDX_W)]],
                    rows_v.at[pl.ds(j * IDX_W, IDX_W)],
                    sem,
                )
                for j in range(GATHERS)
            ]
            for c in copies:
                c.wait()
            pltpu.sync_copy(rows_v, out_hbm.at[pl.ds(row0, CHUNK)])
            return carry

        lax.fori_loop(0, n_iter, body, 0)

    return k


def kernel(cat_ids, table):
    shape = cat_ids.shape
    idx = cat_ids.reshape(-1, 2).astype(jnp.int32)
    idx2 = idx[:, 0] * 12 + idx[:, 1]
    table2 = jnp.concatenate(
        [jnp.repeat(table, 12, axis=0), jnp.tile(table, (12, 1))], axis=1
    )
    out = _gather_call(idx2.size)(table2, idx2)
    return out.reshape(*shape, EMB_DIM)


# speedup vs baseline: 2.7634x; 1.0036x over previous
import functools

import jax
import jax.numpy as jnp
from jax import lax
from jax.experimental import pallas as pl
from jax.experimental.pallas import tpu as pltpu
from jax.experimental.pallas import tpu_sc as plsc

EMB_DIM = 64
IDX_W = 80
GATHERS = 4
CHUNK = IDX_W * GATHERS


@functools.cache
def _gather_call(B2: int):
    info = plsc.get_sparse_core_info()
    NC, NS = info.num_cores, info.num_subcores
    NW = NC * NS
    b_per_w = B2 // NW
    n_iter = b_per_w // CHUNK
    assert b_per_w % CHUNK == 0 and n_iter % 2 == 0
    mesh = plsc.VectorSubcoreMesh(core_axis_name="c", subcore_axis_name="s")

    @functools.partial(
        pl.kernel,
        mesh=mesh,
        out_type=jax.ShapeDtypeStruct((B2, 2 * EMB_DIM), jnp.float32),
        scratch_types=[
            pltpu.VMEM((CHUNK,), jnp.int32),
            pltpu.VMEM((CHUNK,), jnp.int32),
            pltpu.VMEM((CHUNK, 2 * EMB_DIM), jnp.float32),
            pltpu.VMEM((CHUNK, 2 * EMB_DIM), jnp.float32),
            pltpu.SemaphoreType.DMA,
            pltpu.SemaphoreType.DMA,
            pltpu.SemaphoreType.DMA,
            pltpu.SemaphoreType.DMA,
            pltpu.SemaphoreType.DMA,
            pltpu.SemaphoreType.DMA,
        ],
    )
    def k(table2_hbm, idx_hbm, out_hbm,
          idx0, idx1, rows0, rows1, sg0, sg1, so0, so1, si0, si1):
        wid = lax.axis_index("s") * NC + lax.axis_index("c")
        base = wid * b_per_w
        idxb = (idx0, idx1)
        rows = (rows0, rows1)
        sg = (sg0, sg1)
        so = (so0, so1)
        si = (si0, si1)

        def idx_start(i, b):
            pltpu.async_copy(
                idx_hbm.at[pl.ds(base + i * CHUNK, CHUNK)], idxb[b], si[b])

        def idx_wait(b):
            pltpu.make_async_copy(
                idx_hbm.at[pl.ds(base, CHUNK)], idxb[b], si[b]).wait()

        def out_wait(b):
            pltpu.make_async_copy(
                rows[b], out_hbm.at[pl.ds(base, CHUNK)], so[b]).wait()

        idx_start(0, 0)

        def half(io, b, i):
            idx_wait(b)
            @pl.when(io > 0)
            def _():
                out_wait(b)
            copies = [
                pltpu.async_copy(
                    table2_hbm.at[idxb[b].at[pl.ds(j * IDX_W, IDX_W)]],
                    rows[b].at[pl.ds(j * IDX_W, IDX_W)],
                    sg[b],
                )
                for j in range(GATHERS)
            ]
            if b == 0:
                idx_start(i + 1, 1)
            else:
                @pl.when(io < n_iter // 2 - 1)
                def _():
                    idx_start(i + 1, 0)
            for c in copies:
                c.wait()
            pltpu.async_copy(
                rows[b], out_hbm.at[pl.ds(base + i * CHUNK, CHUNK)], so[b])

        def body(io, carry):
            half(io, 0, io * 2)
            half(io, 1, io * 2 + 1)
            return carry

        lax.fori_loop(0, n_iter // 2, body, 0)
        out_wait(0)
        out_wait(1)

    return k


def kernel(cat_ids, table):
    shape = cat_ids.shape
    idx = cat_ids.reshape(-1, 2).astype(jnp.int32)
    idx2 = idx[:, 0] * 12 + idx[:, 1]
    table2 = jnp.concatenate(
        [jnp.repeat(table, 12, axis=0), jnp.tile(table, (12, 1))], axis=1
    )
    out = _gather_call(idx2.size)(table2, idx2)
    return out.reshape(*shape, EMB_DIM)
